# baseline (speedup 1.0000x reference)
import functools

import jax
import jax.numpy as jnp
from jax import lax
from jax.experimental import pallas as pl
from jax.experimental.pallas import tpu as pltpu
from jax.experimental.pallas import tpu_sc as plsc

L = 16
NEG = float("-inf")


def _mm_body(x1_ref, x2_ref, o_ref):
    o_ref[...] = lax.dot_general(
        x1_ref[...], x2_ref[...],
        dimension_numbers=(((1,), (1,)), ((), ())),
        preferred_element_type=jnp.float32,
    )


def _similarity(x1, x2):
    n1 = x1.shape[0]
    n2 = x2.shape[0]
    return pl.pallas_call(
        _mm_body,
        out_shape=jax.ShapeDtypeStruct((n1, n2), jnp.float32),
    )(x1, x2)


def _splat_f32(x):
    return jnp.broadcast_to(jnp.asarray(x, jnp.float32), (L,))


def _greedy_sc(S_flat, N, C):
    nchunk = C // L
    rchunk = N // L
    assert C <= 512, "packed column pairs assume 9-bit column indices"
    info = plsc.get_sparse_core_info()
    rows_per_tile = N // info.num_subcores
    G = min(N, 208)
    mesh = plsc.VectorSubcoreMesh(core_axis_name="c", subcore_axis_name="s")

    @functools.partial(
        pl.kernel,
        mesh=mesh,
        out_type=jax.ShapeDtypeStruct((L,), jnp.float32),
        compiler_params=pltpu.CompilerParams(needs_layout_passes=False),
        scratch_types=[
            pltpu.VMEM_SHARED((N * C,), jnp.float32),
            pltpu.VMEM_SHARED((N,), jnp.float32),
            pltpu.VMEM_SHARED((N,), jnp.int32),
            pltpu.VMEM_SHARED((N,), jnp.float32),
            pltpu.VMEM((G * C,), jnp.float32),
            pltpu.VMEM((N,), jnp.float32),
            pltpu.VMEM((N,), jnp.int32),
            pltpu.VMEM((N,), jnp.float32),
            pltpu.VMEM((C,), jnp.float32),
            pltpu.VMEM((C,), jnp.float32),
            pltpu.VMEM((rows_per_tile,), jnp.float32),
            pltpu.VMEM((rows_per_tile,), jnp.int32),
            pltpu.VMEM((rows_per_tile,), jnp.float32),
            pltpu.VMEM((L,), jnp.float32),
            pltpu.SMEM((N,), jnp.int32),
            pltpu.SMEM((N,), jnp.float32),
            pltpu.SMEM((C,), jnp.int32),
            pltpu.SemaphoreType.DMA,
            pltpu.SemaphoreType.DMA,
            pltpu.SemaphoreType.DMA,
        ],
    )
    def k(S_hbm, out_hbm, S_sh, rep_max, rep_arg, rep_v2, slice_v, rowmax_v,
          argb_v, v2b_v, colmask_v, rowbuf_v, rm_v, ra_v, v2r_v, out_v,
          rowarg_s, v2_s, colalive_s, sem1, sem2, sem3):
        iota = lax.iota(jnp.int32, L)
        cid = lax.axis_index("c")
        sid = lax.axis_index("s")
        base = sid * rows_per_tile

        def write_elem(ref, idx, val):
            off = (idx // L) * L
            vec = ref[pl.ds(off, L)]
            ref[pl.ds(off, L)] = jnp.where(iota == idx - off, val, vec)

        def top2_combine(vb1, vc1, vb2, vc2):
            v1 = jnp.max(vb1)
            c1 = jnp.min(jnp.where(vb1 == v1, vc1 * L + iota, jnp.int32(C)))
            l1mask = iota == (c1 & (L - 1))
            mixed = jnp.where(l1mask, vb2, vb1)
            mixedc = jnp.where(l1mask, vc2, vc1)
            v2 = jnp.max(mixed)
            c2 = jnp.min(jnp.where(mixed == v2, mixedc * L + iota,
                                   jnp.int32(C)))
            return v1, c1, v2, c2

        cp_vmem = pltpu.async_copy(
            S_hbm.at[pl.ds(base * C, rows_per_tile * C)],
            slice_v.at[pl.ds(0, rows_per_tile * C)], sem1)
        cp_spmem = pltpu.async_copy(
            S_hbm.at[pl.ds(base * C, rows_per_tile * C)],
            S_sh.at[pl.ds(base * C, rows_per_tile * C)], sem2)

        @pl.when(sid == 0)
        def _prefetch_cache():
            pltpu.async_copy(
                S_hbm.at[pl.ds(rows_per_tile * C, (G - rows_per_tile) * C)],
                slice_v.at[pl.ds(rows_per_tile * C,
                                 (G - rows_per_tile) * C)], sem3)

        cp_vmem.wait()

        def row_init(l, carry):
            vb1 = jnp.full((L,), NEG, jnp.float32)
            vc1 = jnp.zeros((L,), jnp.int32)
            vb2 = jnp.full((L,), NEG, jnp.float32)
            vc2 = jnp.zeros((L,), jnp.int32)
            rowoff = l * C
            for j in range(nchunk):
                vals = slice_v[pl.ds(rowoff + j * L, L)]
                u1 = vals > vb1
                u2 = vals > vb2
                vb2 = jnp.where(u1, vb1, jnp.where(u2, vals, vb2))
                vc2 = jnp.where(u1, vc1, jnp.where(u2, jnp.int32(j), vc2))
                vb1 = jnp.where(u1, vals, vb1)
                vc1 = jnp.where(u1, jnp.int32(j), vc1)
            v1, c1, v2, c2 = top2_combine(vb1, vc1, vb2, vc2)
            write_elem(rm_v, l, v1)
            write_elem(ra_v, l, c1 | (c2 << 9))
            write_elem(v2r_v, l, v2)
            return carry

        lax.fori_loop(0, rows_per_tile, row_init, jnp.int32(0))
        pltpu.sync_copy(rm_v, rep_max.at[pl.ds(base, rows_per_tile)])
        pltpu.sync_copy(ra_v, rep_arg.at[pl.ds(base, rows_per_tile)])
        pltpu.sync_copy(v2r_v, rep_v2.at[pl.ds(base, rows_per_tile)])
        cp_spmem.wait()
        plsc.subcore_barrier()

        @pl.when(sid == 0)
        def _leader():
            pltpu.make_async_copy(
                S_hbm.at[pl.ds(rows_per_tile * C, (G - rows_per_tile) * C)],
                slice_v.at[pl.ds(rows_per_tile * C,
                                 (G - rows_per_tile) * C)], sem3).wait()
            pltpu.sync_copy(rep_max, rowmax_v)
            pltpu.sync_copy(rep_arg, argb_v)
            pltpu.sync_copy(rep_v2, v2b_v)

            def smem_fill(h, carry):
                veca = argb_v[pl.ds(h * L, L)]
                vecv = v2b_v[pl.ds(h * L, L)]
                for kk in range(L):
                    rowarg_s[h * L + kk] = veca[kk]
                    v2_s[h * L + kk] = vecv[kk]
                    colalive_s[h * L + kk] = jnp.int32(1)
                return carry

            lax.fori_loop(0, rchunk, smem_fill, jnp.int32(0))
            for j in range(nchunk):
                colmask_v[j * L:(j + 1) * L] = jnp.zeros((L,), jnp.float32)

            def init_cm(h):
                cm = jnp.full((L,), NEG, jnp.float32)
                ca = jnp.zeros((L,), jnp.int32)
                for j in range(L):
                    cvec = rowmax_v[(h * L + j) * L:(h * L + j + 1) * L]
                    m = jnp.max(cvec)
                    ln = jnp.min(jnp.where(cvec == m, iota, jnp.int32(L)))
                    cm = jnp.where(iota == j, m, cm)
                    ca = jnp.where(iota == j, ln, ca)
                return cm, ca

            cm0_init, ca0_init = init_cm(0)
            cm1_init, ca1_init = init_cm(1)

            def loop_cond(carry):
                return carry[0] < N

            def loop_body(carry):
                step, tot, cm0, cm1, ca0, ca1 = carry
                mbest = jnp.max(jnp.maximum(cm0, cm1))
                cand = jnp.where(
                    cm0 == mbest, iota,
                    jnp.where(cm1 == mbest, iota + L, jnp.int32(2 * L)))
                chunkv = jnp.broadcast_to(jnp.min(cand), (L,))
                casel = jnp.where(chunkv < L, ca0, ca1)
                laneiv = jnp.where(iota == (chunkv & (L - 1)), casel,
                                   jnp.int32(0))
                rv = chunkv * L + jnp.broadcast_to(jnp.sum(laneiv), (L,))
                r = rv[0]
                chs = r // L
                lanei = r - chs * L
                packed = rowarg_s[r]
                c = packed & jnp.int32(C - 1)
                alive = colalive_s[c] == 1
                rowvec = rowmax_v[pl.ds(chs * L, L)]

                @pl.when(alive)
                def _accept():
                    rowmax_v[pl.ds(chs * L, L)] = jnp.where(
                        iota == lanei, jnp.float32(NEG), rowvec)
                    colalive_s[c] = jnp.int32(0)
                    coff = (c // L) * L
                    cvec = colmask_v[pl.ds(coff, L)]
                    colmask_v[pl.ds(coff, L)] = jnp.where(
                        iota == c - coff, jnp.float32(NEG), cvec)

                @pl.when(jnp.logical_not(alive))
                def _stale():
                    c2 = packed >> 9
                    alive2 = colalive_s[c2] == 1

                    @pl.when(alive2)
                    def _promote():
                        v2 = v2_s[r]
                        rowmax_v[pl.ds(chs * L, L)] = jnp.where(
                            iota == lanei, v2, rowvec)
                        rowarg_s[r] = c2 | (c << 9)

                    def rescan_with(load_chunk):
                        vb1 = jnp.full((L,), NEG, jnp.float32)
                        vc1 = jnp.zeros((L,), jnp.int32)
                        vb2 = jnp.full((L,), NEG, jnp.float32)
                        vc2 = jnp.zeros((L,), jnp.int32)
                        for j in range(nchunk):
                            vals = (load_chunk(j)
                                    + colmask_v[j * L:(j + 1) * L])
                            u1 = vals > vb1
                            u2 = vals > vb2
                            vb2 = jnp.where(u1, vb1,
                                            jnp.where(u2, vals, vb2))
                            vc2 = jnp.where(u1, vc1,
                                            jnp.where(u2, jnp.int32(j), vc2))
                            vb1 = jnp.where(u1, vals, vb1)
                            vc1 = jnp.where(u1, jnp.int32(j), vc1)
                        v1n, c1n, v2n, c2n = top2_combine(vb1, vc1, vb2, vc2)
                        rowmax_v[pl.ds(chs * L, L)] = jnp.where(
                            iota == lanei, v1n, rowvec)
                        rowarg_s[r] = c1n | (c2n << 9)
                        v2_s[r] = v2n

                    @pl.when(jnp.logical_not(alive2) & (r < G))
                    def _rescan_local():
                        rescan_with(
                            lambda j: slice_v[pl.ds(r * C + j * L, L)])

                    @pl.when(jnp.logical_not(alive2) & (r >= G))
                    def _rescan_remote():
                        pltpu.sync_copy(S_sh.at[pl.ds(r * C, C)], rowbuf_v)
                        rescan_with(
                            lambda j: rowbuf_v[j * L:(j + 1) * L])

                chunkvec = rowmax_v[pl.ds(chs * L, L)]
                newmax = jnp.max(chunkvec)
                newlane = jnp.min(
                    jnp.where(chunkvec == newmax, iota, jnp.int32(L)))
                inhalf0 = chs < L
                lanec = chs - jnp.where(inhalf0, jnp.int32(0), jnp.int32(L))
                upd_mask = iota == lanec
                m0 = jnp.broadcast_to(inhalf0, (L,)) & upd_mask
                m1 = jnp.broadcast_to(~inhalf0, (L,)) & upd_mask
                cm0n = jnp.where(m0, newmax, cm0)
                cm1n = jnp.where(m1, newmax, cm1)
                ca0n = jnp.where(m0, newlane, ca0)
                ca1n = jnp.where(m1, newlane, ca1)
                stepn = step + alive.astype(jnp.int32)
                totn = tot + jnp.where(alive, mbest, jnp.float32(0.0))
                return (stepn, totn, cm0n, cm1n, ca0n, ca1n)

            _, tot, _, _, _, _ = lax.while_loop(
                loop_cond, loop_body,
                (jnp.int32(0), jnp.float32(0.0), cm0_init, cm1_init,
                 ca0_init, ca1_init))

            out_v[...] = _splat_f32(tot * jnp.float32(1.0 / max(N, C)))

            @pl.when(cid == 0)
            def _write():
                pltpu.sync_copy(out_v, out_hbm)

    return k(S_flat)


def kernel(x1, x2):
    S = _similarity(x1, x2)
    out = _greedy_sc(S.reshape(-1), S.shape[0], S.shape[1])
    return out[0]


# speedup vs baseline: 1.0798x; 1.0798x over previous
import functools

import jax
import jax.numpy as jnp
from jax import lax
from jax.experimental import pallas as pl
from jax.experimental.pallas import tpu as pltpu
from jax.experimental.pallas import tpu_sc as plsc

L = 16
NEG = float("-inf")


def _mm_body(x1_ref, x2_ref, o_ref):
    o_ref[...] = lax.dot_general(
        x1_ref[...], x2_ref[...],
        dimension_numbers=(((1,), (1,)), ((), ())),
        preferred_element_type=jnp.float32,
    )


def _similarity(x1, x2):
    n1 = x1.shape[0]
    n2 = x2.shape[0]
    return pl.pallas_call(
        _mm_body,
        out_shape=jax.ShapeDtypeStruct((n1, n2), jnp.float32),
    )(x1, x2)


def _splat_f32(x):
    return jnp.broadcast_to(jnp.asarray(x, jnp.float32), (L,))


def _greedy_sc(S_flat, N, C):
    nchunk = C // L
    rchunk = N // L
    assert C <= 512, "packed column pairs assume 9-bit column indices"
    info = plsc.get_sparse_core_info()
    rows_per_tile = N // info.num_subcores
    G = min(N, 208)
    mesh = plsc.VectorSubcoreMesh(core_axis_name="c", subcore_axis_name="s")

    @functools.partial(
        pl.kernel,
        mesh=mesh,
        out_type=jax.ShapeDtypeStruct((L,), jnp.float32),
        compiler_params=pltpu.CompilerParams(needs_layout_passes=False),
        scratch_types=[
            pltpu.VMEM_SHARED((N * C,), jnp.float32),
            pltpu.VMEM_SHARED((N,), jnp.float32),
            pltpu.VMEM_SHARED((N,), jnp.int32),
            pltpu.VMEM_SHARED((N,), jnp.float32),
            pltpu.VMEM((G * C,), jnp.float32),
            pltpu.VMEM((N,), jnp.float32),
            pltpu.VMEM((N,), jnp.int32),
            pltpu.VMEM((N,), jnp.float32),
            pltpu.VMEM((C,), jnp.float32),
            pltpu.VMEM((C,), jnp.float32),
            pltpu.VMEM((rows_per_tile,), jnp.float32),
            pltpu.VMEM((rows_per_tile,), jnp.int32),
            pltpu.VMEM((rows_per_tile,), jnp.float32),
            pltpu.VMEM((L,), jnp.float32),
            pltpu.SMEM((N,), jnp.int32),
            pltpu.SMEM((N,), jnp.float32),
            pltpu.SMEM((C,), jnp.int32),
            pltpu.SemaphoreType.DMA,
            pltpu.SemaphoreType.DMA,
            pltpu.SemaphoreType.DMA,
        ],
    )
    def k(S_hbm, out_hbm, S_sh, rep_max, rep_arg, rep_v2, slice_v, rowmax_v,
          argb_v, v2b_v, colmask_v, rowbuf_v, rm_v, ra_v, v2r_v, out_v,
          rowarg_s, v2_s, colalive_s, sem1, sem2, sem3):
        iota = lax.iota(jnp.int32, L)
        cid = lax.axis_index("c")
        sid = lax.axis_index("s")
        base = sid * rows_per_tile

        def write_elem(ref, idx, val):
            off = (idx // L) * L
            vec = ref[pl.ds(off, L)]
            ref[pl.ds(off, L)] = jnp.where(iota == idx - off, val, vec)

        def top2_combine(vb1, vc1, vb2, vc2):
            v1 = jnp.max(vb1)
            c1 = jnp.min(jnp.where(vb1 == v1, vc1 * L + iota, jnp.int32(C)))
            l1mask = iota == (c1 & (L - 1))
            mixed = jnp.where(l1mask, vb2, vb1)
            mixedc = jnp.where(l1mask, vc2, vc1)
            v2 = jnp.max(mixed)
            c2 = jnp.min(jnp.where(mixed == v2, mixedc * L + iota,
                                   jnp.int32(C)))
            return v1, c1, v2, c2

        cp_vmem = pltpu.async_copy(
            S_hbm.at[pl.ds(base * C, rows_per_tile * C)],
            slice_v.at[pl.ds(0, rows_per_tile * C)], sem1)
        cp_spmem = pltpu.async_copy(
            S_hbm.at[pl.ds(base * C, rows_per_tile * C)],
            S_sh.at[pl.ds(base * C, rows_per_tile * C)], sem2)

        @pl.when(sid == 0)
        def _prefetch_cache():
            pltpu.async_copy(
                S_hbm.at[pl.ds(rows_per_tile * C, (G - rows_per_tile) * C)],
                slice_v.at[pl.ds(rows_per_tile * C,
                                 (G - rows_per_tile) * C)], sem3)

        cp_vmem.wait()

        def row_init(l, carry):
            vb1 = jnp.full((L,), NEG, jnp.float32)
            vc1 = jnp.zeros((L,), jnp.int32)
            vb2 = jnp.full((L,), NEG, jnp.float32)
            vc2 = jnp.zeros((L,), jnp.int32)
            rowoff = l * C
            for j in range(nchunk):
                vals = slice_v[pl.ds(rowoff + j * L, L)]
                u1 = vals > vb1
                u2 = vals > vb2
                vb2 = jnp.where(u1, vb1, jnp.where(u2, vals, vb2))
                vc2 = jnp.where(u1, vc1, jnp.where(u2, jnp.int32(j), vc2))
                vb1 = jnp.where(u1, vals, vb1)
                vc1 = jnp.where(u1, jnp.int32(j), vc1)
            v1, c1, v2, c2 = top2_combine(vb1, vc1, vb2, vc2)
            write_elem(rm_v, l, v1)
            write_elem(ra_v, l, c1 | (c2 << 9))
            write_elem(v2r_v, l, v2)
            return carry

        lax.fori_loop(0, rows_per_tile, row_init, jnp.int32(0))
        pltpu.sync_copy(rm_v, rep_max.at[pl.ds(base, rows_per_tile)])
        pltpu.sync_copy(ra_v, rep_arg.at[pl.ds(base, rows_per_tile)])
        pltpu.sync_copy(v2r_v, rep_v2.at[pl.ds(base, rows_per_tile)])
        cp_spmem.wait()
        plsc.subcore_barrier()

        @pl.when(sid == 0)
        def _leader():
            pltpu.make_async_copy(
                S_hbm.at[pl.ds(rows_per_tile * C, (G - rows_per_tile) * C)],
                slice_v.at[pl.ds(rows_per_tile * C,
                                 (G - rows_per_tile) * C)], sem3).wait()
            pltpu.sync_copy(rep_max, rowmax_v)
            pltpu.sync_copy(rep_arg, argb_v)
            pltpu.sync_copy(rep_v2, v2b_v)

            def smem_fill(h, carry):
                veca = argb_v[pl.ds(h * L, L)]
                vecv = v2b_v[pl.ds(h * L, L)]
                for kk in range(L):
                    rowarg_s[h * L + kk] = veca[kk]
                    v2_s[h * L + kk] = vecv[kk]
                    colalive_s[h * L + kk] = jnp.int32(1)
                return carry

            lax.fori_loop(0, rchunk, smem_fill, jnp.int32(0))
            for j in range(nchunk):
                colmask_v[j * L:(j + 1) * L] = jnp.zeros((L,), jnp.float32)

            def init_cm(h):
                cm = jnp.full((L,), NEG, jnp.float32)
                for j in range(L):
                    m = jnp.max(rowmax_v[(h * L + j) * L:(h * L + j + 1) * L])
                    cm = jnp.where(iota == j, m, cm)
                return cm

            cm0_init = init_cm(0)
            cm1_init = init_cm(1)
            mb_init = jnp.max(jnp.maximum(cm0_init, cm1_init))

            def loop_cond(carry):
                return carry[0] < N

            def loop_body(carry):
                step, tot, mbest, cm0, cm1 = carry
                cand = jnp.where(
                    cm0 == mbest, iota,
                    jnp.where(cm1 == mbest, iota + L, jnp.int32(2 * L)))
                chunk = jnp.min(cand)
                chunkb = jnp.broadcast_to(chunk, (L,))
                updm = iota == (chunkb & (L - 1))
                m0 = (chunkb < L) & updm
                m1 = (chunkb >= L) & updm
                negv = jnp.full((L,), NEG, jnp.float32)
                m2 = jnp.max(jnp.maximum(jnp.where(m0, negv, cm0),
                                         jnp.where(m1, negv, cm1)))
                rowvec = rowmax_v[pl.ds(chunk * L, L)]
                lanei = jnp.min(
                    jnp.where(rowvec == mbest, iota, jnp.int32(L)))
                r = chunk * L + lanei
                packed = rowarg_s[r]
                c = packed & jnp.int32(C - 1)
                alive = colalive_s[c] == 1

                @pl.when(alive)
                def _accept():
                    rowmax_v[pl.ds(chunk * L, L)] = jnp.where(
                        iota == lanei, jnp.float32(NEG), rowvec)
                    colalive_s[c] = jnp.int32(0)
                    coff = (c // L) * L
                    cvec = colmask_v[pl.ds(coff, L)]
                    colmask_v[pl.ds(coff, L)] = jnp.where(
                        iota == c - coff, jnp.float32(NEG), cvec)

                @pl.when(jnp.logical_not(alive))
                def _stale():
                    c2 = packed >> 9
                    alive2 = colalive_s[c2] == 1

                    @pl.when(alive2)
                    def _promote():
                        v2 = v2_s[r]
                        rowmax_v[pl.ds(chunk * L, L)] = jnp.where(
                            iota == lanei, v2, rowvec)
                        rowarg_s[r] = c2 | (c << 9)

                    def rescan_with(load_chunk):
                        vb1 = jnp.full((L,), NEG, jnp.float32)
                        vc1 = jnp.zeros((L,), jnp.int32)
                        vb2 = jnp.full((L,), NEG, jnp.float32)
                        vc2 = jnp.zeros((L,), jnp.int32)
                        for j in range(nchunk):
                            vals = (load_chunk(j)
                                    + colmask_v[j * L:(j + 1) * L])
                            u1 = vals > vb1
                            u2 = vals > vb2
                            vb2 = jnp.where(u1, vb1,
                                            jnp.where(u2, vals, vb2))
                            vc2 = jnp.where(u1, vc1,
                                            jnp.where(u2, jnp.int32(j), vc2))
                            vb1 = jnp.where(u1, vals, vb1)
                            vc1 = jnp.where(u1, jnp.int32(j), vc1)
                        v1n, c1n, v2n, c2n = top2_combine(vb1, vc1, vb2, vc2)
                        rowmax_v[pl.ds(chunk * L, L)] = jnp.where(
                            iota == lanei, v1n, rowvec)
                        rowarg_s[r] = c1n | (c2n << 9)
                        v2_s[r] = v2n

                    @pl.when(jnp.logical_not(alive2) & (r < G))
                    def _rescan_local():
                        rescan_with(
                            lambda j: slice_v[pl.ds(r * C + j * L, L)])

                    @pl.when(jnp.logical_not(alive2) & (r >= G))
                    def _rescan_remote():
                        pltpu.sync_copy(S_sh.at[pl.ds(r * C, C)], rowbuf_v)
                        rescan_with(
                            lambda j: rowbuf_v[j * L:(j + 1) * L])

                chunkvec = rowmax_v[pl.ds(chunk * L, L)]
                newmax = jnp.max(chunkvec)
                cm0n = jnp.where(m0, newmax, cm0)
                cm1n = jnp.where(m1, newmax, cm1)
                mbn = jnp.maximum(m2, newmax)
                stepn = step + alive.astype(jnp.int32)
                totn = tot + jnp.where(alive, mbest, jnp.float32(0.0))
                return (stepn, totn, mbn, cm0n, cm1n)

            _, tot, _, _, _ = lax.while_loop(
                loop_cond, loop_body,
                (jnp.int32(0), jnp.float32(0.0), mb_init, cm0_init,
                 cm1_init))

            out_v[...] = _splat_f32(tot * jnp.float32(1.0 / max(N, C)))

            @pl.when(cid == 0)
            def _write():
                pltpu.sync_copy(out_v, out_hbm)

    return k(S_flat)


def kernel(x1, x2):
    S = _similarity(x1, x2)
    out = _greedy_sc(S.reshape(-1), S.shape[0], S.shape[1])
    return out[0]
